# baseline (speedup 1.0000x reference)
import jax
import jax.numpy as jnp
from jax.experimental import pallas as pl

NPTS = 4096
N = 16384
B = 8
LANES = 128


def _fps_kernel(x_ref, y_ref, z_ref, idx_ref, ox_ref, oy_ref, oz_ref):
    x = x_ref[...]
    y = y_ref[...]
    z = z_ref[...]

    nchunks = x.shape[1] // LANES
    xs = [x[:, k * LANES:(k + 1) * LANES] for k in range(nchunks)]
    ys = [y[:, k * LANES:(k + 1) * LANES] for k in range(nchunks)]
    zs = [z[:, k * LANES:(k + 1) * LANES] for k in range(nchunks)]
    lane_iota = jax.lax.broadcasted_iota(jnp.int32, (x.shape[0], LANES), 1)
    iotas = [lane_iota + jnp.int32(k * LANES) for k in range(nchunks)]

    idx_ref[0:1, :] = jnp.zeros((1, x.shape[0]), jnp.int32)
    ox_ref[0:1, :] = x[:, 0:1].T
    oy_ref[0:1, :] = y[:, 0:1].T
    oz_ref[0:1, :] = z[:, 0:1].T

    dists0 = jnp.full(x.shape, 1e10, dtype=jnp.float32)
    c0 = (x[:, 0:1], y[:, 0:1], z[:, 0:1])

    def body(i, carry):
        dists, cx, cy, cz = carry
        dx = x - cx
        dy = y - cy
        dz = z - cz
        d = (dx * dx + dz * dz) + dy * dy
        new_dists = jnp.minimum(dists, d)
        nodes = [
            (new_dists[:, k * LANES:(k + 1) * LANES],
             iotas[k], xs[k], ys[k], zs[k])
            for k in range(nchunks)
        ]
        while len(nodes) > 1:
            nxt_nodes = []
            for k in range(0, len(nodes), 2):
                a, b = nodes[k], nodes[k + 1]
                take_a = a[0] >= b[0]
                nxt_nodes.append(tuple(
                    jnp.where(take_a, fa, fb) for fa, fb in zip(a, b)))
            nodes = nxt_nodes
        v, iw, xw, yw, zw = nodes[0]
        m = jnp.max(v, axis=1, keepdims=True)
        nxt = jnp.min(
            jnp.where(v == m, iw, jnp.int32(N)), axis=1, keepdims=True
        )
        onehot = iw == nxt
        ncx = jnp.sum(jnp.where(onehot, xw, 0.0), axis=1, keepdims=True)
        ncy = jnp.sum(jnp.where(onehot, yw, 0.0), axis=1, keepdims=True)
        ncz = jnp.sum(jnp.where(onehot, zw, 0.0), axis=1, keepdims=True)
        idx_ref[pl.ds(i, 1), :] = nxt.T
        ox_ref[pl.ds(i, 1), :] = ncx.T
        oy_ref[pl.ds(i, 1), :] = ncy.T
        oz_ref[pl.ds(i, 1), :] = ncz.T
        return (new_dists, ncx, ncy, ncz)

    jax.lax.fori_loop(1, idx_ref.shape[0], body, (dists0,) + c0)


@jax.jit
def kernel(inp):
    x = inp[:, :, 0]
    y = inp[:, :, 1]
    z = inp[:, :, 2]
    out_types = (
        jax.ShapeDtypeStruct((NPTS, B), jnp.int32),
        jax.ShapeDtypeStruct((NPTS, B), jnp.float32),
        jax.ShapeDtypeStruct((NPTS, B), jnp.float32),
        jax.ShapeDtypeStruct((NPTS, B), jnp.float32),
    )
    idx_t, ox_t, oy_t, oz_t = pl.pallas_call(
        _fps_kernel,
        out_shape=out_types,
        in_specs=[
            pl.BlockSpec((B, N), lambda: (0, 0)),
            pl.BlockSpec((B, N), lambda: (0, 0)),
            pl.BlockSpec((B, N), lambda: (0, 0)),
        ],
        out_specs=(
            pl.BlockSpec((NPTS, B), lambda: (0, 0)),
            pl.BlockSpec((NPTS, B), lambda: (0, 0)),
            pl.BlockSpec((NPTS, B), lambda: (0, 0)),
            pl.BlockSpec((NPTS, B), lambda: (0, 0)),
        ),
    )(x, y, z)
    idx = idx_t.T
    out = jnp.stack([ox_t.T, oy_t.T, oz_t.T], axis=-1)
    return (out, idx)


# speedup vs baseline: 1.1910x; 1.1910x over previous
import jax
import jax.numpy as jnp
from jax.experimental import pallas as pl

NPTS = 4096
N = 16384
B = 8
LANES = 128


def _fps_kernel(x_ref, y_ref, z_ref, idx_ref, ox_ref, oy_ref, oz_ref):
    x = x_ref[...]
    y = y_ref[...]
    z = z_ref[...]

    nchunks = x.shape[1] // LANES
    xs = [x[:, k * LANES:(k + 1) * LANES] for k in range(nchunks)]
    ys = [y[:, k * LANES:(k + 1) * LANES] for k in range(nchunks)]
    zs = [z[:, k * LANES:(k + 1) * LANES] for k in range(nchunks)]
    lane_iota = jax.lax.broadcasted_iota(jnp.int32, (x.shape[0], LANES), 1)
    iotas = [lane_iota + jnp.int32(k * LANES) for k in range(nchunks)]

    idx_ref[0:1, :] = jnp.zeros((1, x.shape[0]), jnp.int32)
    ox_ref[0:1, :] = x[:, 0:1].T
    oy_ref[0:1, :] = y[:, 0:1].T
    oz_ref[0:1, :] = z[:, 0:1].T

    dists0 = jnp.full(x.shape, 1e10, dtype=jnp.float32)
    c0 = (x[:, 0:1], y[:, 0:1], z[:, 0:1])

    def body(i, carry):
        dists, cx, cy, cz = carry
        nodes = []
        for k in range(nchunks):
            dx = xs[k] - cx
            dy = ys[k] - cy
            dz = zs[k] - cz
            d = (dx * dx + dz * dz) + dy * dy
            v = jnp.minimum(dists[:, k * LANES:(k + 1) * LANES], d)
            nodes.append((v, iotas[k], xs[k], ys[k], zs[k]))
        new_dists = jnp.concatenate([n[0] for n in nodes], axis=1)
        while len(nodes) > 1:
            nxt_nodes = []
            for k in range(0, len(nodes), 2):
                a, b = nodes[k], nodes[k + 1]
                take_a = a[0] >= b[0]
                nxt_nodes.append(tuple(
                    jnp.where(take_a, fa, fb) for fa, fb in zip(a, b)))
            nodes = nxt_nodes
        v, iw, xw, yw, zw = nodes[0]
        m = jnp.max(v, axis=1, keepdims=True)
        nxt = jnp.min(
            jnp.where(v == m, iw, jnp.int32(N)), axis=1, keepdims=True
        )
        onehot = iw == nxt
        ncx = jnp.sum(jnp.where(onehot, xw, 0.0), axis=1, keepdims=True)
        ncy = jnp.sum(jnp.where(onehot, yw, 0.0), axis=1, keepdims=True)
        ncz = jnp.sum(jnp.where(onehot, zw, 0.0), axis=1, keepdims=True)
        idx_ref[pl.ds(i, 1), :] = nxt.T
        ox_ref[pl.ds(i, 1), :] = ncx.T
        oy_ref[pl.ds(i, 1), :] = ncy.T
        oz_ref[pl.ds(i, 1), :] = ncz.T
        return (new_dists, ncx, ncy, ncz)

    jax.lax.fori_loop(1, idx_ref.shape[0], body, (dists0,) + c0)


@jax.jit
def kernel(inp):
    x = inp[:, :, 0]
    y = inp[:, :, 1]
    z = inp[:, :, 2]
    out_types = (
        jax.ShapeDtypeStruct((NPTS, B), jnp.int32),
        jax.ShapeDtypeStruct((NPTS, B), jnp.float32),
        jax.ShapeDtypeStruct((NPTS, B), jnp.float32),
        jax.ShapeDtypeStruct((NPTS, B), jnp.float32),
    )
    idx_t, ox_t, oy_t, oz_t = pl.pallas_call(
        _fps_kernel,
        out_shape=out_types,
        in_specs=[
            pl.BlockSpec((B, N), lambda: (0, 0)),
            pl.BlockSpec((B, N), lambda: (0, 0)),
            pl.BlockSpec((B, N), lambda: (0, 0)),
        ],
        out_specs=(
            pl.BlockSpec((NPTS, B), lambda: (0, 0)),
            pl.BlockSpec((NPTS, B), lambda: (0, 0)),
            pl.BlockSpec((NPTS, B), lambda: (0, 0)),
            pl.BlockSpec((NPTS, B), lambda: (0, 0)),
        ),
    )(x, y, z)
    idx = idx_t.T
    out = jnp.stack([ox_t.T, oy_t.T, oz_t.T], axis=-1)
    return (out, idx)
